# Initial kernel scaffold; baseline (speedup 1.0000x reference)
import jax
import jax.numpy as jnp
from jax.experimental import pallas as pl


def kernel(prev_outputs, prev_is_active, parent_indices, w1, b1, w2, b2):
    raise NotImplementedError("write your pallas kernel here")


# speedup vs baseline: 10.3641x; 10.3641x over previous
import functools

import jax
import jax.numpy as jnp
import numpy as np
from jax.experimental import pallas as pl
from jax.experimental.pallas import tpu as pltpu

M = 4096
K = 2048
N = 16
FAN_IN = 16
TOP_K = 12
INTERACT = 8
PRESERVE = 4
ACT_THRESHOLD = 8
HIDDEN = 16
_IDX1, _IDX2 = np.triu_indices(INTERACT, k=1)
N_PAIRS = len(_IDX1)
IN_CH = N_PAIRS + PRESERVE

KB = 16
_NEG_INACTIVE = -1e30
_NEG_TAKEN = -2e30


def _slogabsdet_16(A):
    b = A.shape[0]
    rowids = jax.lax.broadcasted_iota(jnp.int32, (b, N), 1)
    logdet = jnp.zeros((b,), jnp.float32)
    for kcol in range(N):
        col = A[:, :, kcol]
        abscol = jnp.where(rowids >= kcol, jnp.abs(col), -1.0)
        piv = jnp.argmax(abscol, axis=1)
        oh_p = (rowids == piv[:, None]).astype(jnp.float32)
        oh_k = (rowids == kcol).astype(jnp.float32)
        row_p = jnp.sum(oh_p[:, :, None] * A, axis=1)
        row_k = A[:, kcol, :]
        A = (A * (1.0 - oh_k - oh_p)[:, :, None]
             + oh_k[:, :, None] * row_p[:, None, :]
             + oh_p[:, :, None] * row_k[:, None, :])
        pivval = row_p[:, kcol]
        logdet = logdet + jnp.log(jnp.abs(pivval))
        safe_piv = jnp.where(pivval == 0.0, 1.0, pivval)
        mult = jnp.where(rowids > kcol, A[:, :, kcol] / safe_piv[:, None], 0.0)
        A = A - mult[:, :, None] * A[:, kcol, :][:, None, :]
    return logdet


def _decoder_block_kernel(g_ref, fl_ref, weff_ref, beff_ref,
                          out_ref, gate_ref):
    g = g_ref[...]
    flags = fl_ref[...]
    weff = weff_ref[...]
    beff = beff_ref[...]

    logdet = _slogabsdet_16(g.reshape(KB * FAN_IN, N, N)).reshape(KB, FAN_IN)
    scores = jnp.where(flags > 0.5, logdet, _NEG_INACTIVE)

    colids = jax.lax.broadcasted_iota(jnp.int32, (KB, FAN_IN), 1)
    avail = jnp.ones((KB, FAN_IN), jnp.bool_)
    top = []
    for _ in range(TOP_K):
        sc = jnp.where(avail, scores, _NEG_TAKEN)
        am = jnp.argmax(sc, axis=1)
        oh = colids == am[:, None]
        avail = jnp.logical_and(avail, jnp.logical_not(oh))
        ohf = oh.astype(jnp.float32)
        sel = jnp.sum(ohf[:, :, None, None] * g, axis=1)
        top.append(sel)

    pre = jnp.zeros((KB, N, N), jnp.float32) + beff[:, :, None]
    for p, (i, j) in enumerate(zip(_IDX1, _IDX2)):
        L, R = top[i], top[j]
        acc = L[:, :, 0][:, :, None] * R[:, 0, :][:, None, :]
        for e in range(1, N):
            acc = acc + L[:, :, e][:, :, None] * R[:, e, :][:, None, :]
        pre = pre + weff[:, p][:, None, None] * acc
    for q in range(PRESERVE):
        pre = pre + weff[:, N_PAIRS + q][:, None, None] * top[INTERACT + q]

    act = pre * jax.nn.sigmoid(pre)
    gate_f = (jnp.sum(flags, axis=1) >= float(ACT_THRESHOLD)).astype(jnp.float32)
    out_ref[...] = act * gate_f[:, None, None]
    gate_ref[...] = gate_f[:, None].astype(jnp.int32)


@functools.partial(jax.jit, static_argnums=())
def kernel(prev_outputs, prev_is_active, parent_indices, w1, b1, w2, b2):
    gathered = prev_outputs[parent_indices]
    flags = prev_is_active[parent_indices].astype(jnp.float32)

    w2f = w2.reshape(K, HIDDEN)
    b1f = b1.reshape(K, HIDDEN)
    weff = jnp.einsum('ko,koc->kc', w2f, w1)
    beff = (jnp.sum(w2f * b1f, axis=1, keepdims=True)
            + b2.reshape(K, 1))

    grid = (K // KB,)
    out, gate = pl.pallas_call(
        _decoder_block_kernel,
        grid=grid,
        in_specs=[
            pl.BlockSpec((KB, FAN_IN, N, N), lambda i: (i, 0, 0, 0)),
            pl.BlockSpec((KB, FAN_IN), lambda i: (i, 0)),
            pl.BlockSpec((KB, IN_CH), lambda i: (i, 0)),
            pl.BlockSpec((KB, 1), lambda i: (i, 0)),
        ],
        out_specs=[
            pl.BlockSpec((KB, N, N), lambda i: (i, 0, 0)),
            pl.BlockSpec((KB, 1), lambda i: (i, 0)),
        ],
        out_shape=[
            jax.ShapeDtypeStruct((K, N, N), jnp.float32),
            jax.ShapeDtypeStruct((K, 1), jnp.int32),
        ],
        compiler_params=pltpu.CompilerParams(
            dimension_semantics=("arbitrary",),
        ),
    )(gathered, flags, weff, beff)

    return out, gate[:, 0].astype(jnp.bool_)
